# Initial kernel scaffold; baseline (speedup 1.0000x reference)
import jax
import jax.numpy as jnp
from jax.experimental import pallas as pl


def kernel(xyz, cls_label, W1, b1, W2, b2, W3, b3, Wf3, bf3, Wf2, bf2, Wf1, bf1, Wc, bc):
    raise NotImplementedError("write your pallas kernel here")


# speedup vs baseline: 4.0322x; 4.0322x over previous
import functools

import jax
import jax.numpy as jnp
from jax.experimental import pallas as pl
from jax.experimental.pallas import tpu as pltpu

F32 = jnp.float32
NEG = -1e30


def _fps_body(xyz_ref, out_ref, dist_ref, *, npoint, n):
    x = xyz_ref[:, 0, :]
    y = xyz_ref[:, 1, :]
    z = xyz_ref[:, 2, :]
    b = x.shape[0]
    dist_ref[:, :] = jnp.full((b, n), 1e10, F32)
    iota = jax.lax.broadcasted_iota(jnp.int32, (b, n), 1)

    def step(i, carry):
        cx, cy, cz = carry
        out_ref[:, pl.ds(i, 1), 0:1] = cx[:, :, None]
        out_ref[:, pl.ds(i, 1), 1:2] = cy[:, :, None]
        out_ref[:, pl.ds(i, 1), 2:3] = cz[:, :, None]
        d = (x - cx) ** 2 + (y - cy) ** 2 + (z - cz) ** 2
        dist = jnp.minimum(dist_ref[:, :], d)
        dist_ref[:, :] = dist
        m = jnp.max(dist, axis=1, keepdims=True)
        ii = jnp.min(jnp.where(dist == m, iota, n), axis=1, keepdims=True)
        oh = iota == ii
        ncx = jnp.sum(jnp.where(oh, x, 0.0), axis=1, keepdims=True)
        ncy = jnp.sum(jnp.where(oh, y, 0.0), axis=1, keepdims=True)
        ncz = jnp.sum(jnp.where(oh, z, 0.0), axis=1, keepdims=True)
        return ncx, ncy, ncz

    c0 = (x[:, 0:1], y[:, 0:1], z[:, 0:1])
    jax.lax.fori_loop(0, npoint, step, c0)


def _fps(xyz_planes, npoint):
    b, _, n = xyz_planes.shape
    return pl.pallas_call(
        functools.partial(_fps_body, npoint=npoint, n=n),
        out_shape=jax.ShapeDtypeStruct((b, npoint, 3), F32),
        scratch_shapes=[pltpu.VMEM((b, n), F32)],
    )(xyz_planes)


def _sa_chunk_step(xyz4_ref, cenc_ref, b_ref, wa_ref, out_ref, f_ref,
                   macc_ref, off_ref, *, nchunk, nsample, r2):
    k = pl.program_id(2)
    cen = cenc_ref[0]
    s = cen.shape[0]
    cdim = f_ref.shape[1]

    @pl.when(k == 0)
    def _():
        macc_ref[:, :] = jnp.full((s, cdim), NEG, F32)
        off_ref[:, :] = jnp.zeros((s, 1), F32)

    cx = cen[:, 0:1]
    cy = cen[:, 1:2]
    cz = cen[:, 2:3]
    rit = jax.lax.broadcasted_iota(jnp.int32, (128, 128), 0)
    cit = jax.lax.broadcasted_iota(jnp.int32, (128, 128), 1)
    ut = (rit <= cit).astype(F32)

    x = xyz4_ref[0, 0, 0, 0:1, :]
    y = xyz4_ref[0, 1, 0, 0:1, :]
    z = xyz4_ref[0, 2, 0, 0:1, :]
    d = (cx - x) ** 2 + (cy - y) ** 2 + (cz - z) ** 2
    inb = d <= r2
    pc = (jnp.dot(inb.astype(F32), ut, preferred_element_type=F32)
          + off_ref[:, :])
    sel = jnp.logical_and(inb, pc <= float(nsample))
    fc = f_ref[pl.ds(k * 128, 128), :]
    terms = [jnp.where(sel[:, j:j + 1], fc[j:j + 1, :], NEG)
             for j in range(128)]
    while len(terms) > 1:
        terms = [jnp.maximum(terms[2 * i], terms[2 * i + 1])
                 for i in range(len(terms) // 2)]
    macc_ref[:, :] = jnp.maximum(macc_ref[:, :], terms[0])
    off_ref[:, :] = pc[:, 127:128]

    @pl.when(k == nchunk - 1)
    def _():
        a = jnp.dot(cen, wa_ref[:, :], preferred_element_type=F32)
        out_ref[0] = jnp.maximum(macc_ref[:, :] - a + b_ref[0:1, :], 0.0)


def _sa1_body(xyz4_ref, xyzc_ref, cenc_ref, wsum_ref, wa_ref, b_ref,
              out_ref, f_ref, macc_ref, off_ref, *, nchunk, nsample, r2):
    @pl.when(jnp.logical_and(pl.program_id(1) == 0, pl.program_id(2) == 0))
    def _():
        f_ref[:, :] = jnp.dot(xyzc_ref[0], wsum_ref[:, :],
                              preferred_element_type=F32)

    _sa_chunk_step(xyz4_ref, cenc_ref, b_ref, wa_ref, out_ref, f_ref,
                   macc_ref, off_ref, nchunk=nchunk, nsample=nsample, r2=r2)


def _sa2_body(xyz4_ref, xyzc_ref, cenc_ref, pts_ref, wa_ref, wp_ref, b_ref,
              out_ref, f_ref, macc_ref, off_ref, *, nchunk, nsample, r2):
    @pl.when(jnp.logical_and(pl.program_id(1) == 0, pl.program_id(2) == 0))
    def _():
        f_ref[:, :] = (
            jnp.dot(xyzc_ref[0], wa_ref[:, :], preferred_element_type=F32)
            + jnp.dot(pts_ref[0], wp_ref[:, :], preferred_element_type=F32))

    _sa_chunk_step(xyz4_ref, cenc_ref, b_ref, wa_ref, out_ref, f_ref,
                   macc_ref, off_ref, nchunk=nchunk, nsample=nsample, r2=r2)


def _sa1(xyz4, xyzc, cenc, wsum, wa, bvec, *, nsample, r2, s_tile):
    b, n3, nchunk = xyz4.shape[:3]
    n = nchunk * 128
    s_total = cenc.shape[1]
    cdim = wa.shape[1]
    return pl.pallas_call(
        functools.partial(_sa1_body, nchunk=nchunk, nsample=nsample, r2=r2),
        grid=(b, s_total // s_tile, nchunk),
        in_specs=[
            pl.BlockSpec((1, 3, 1, 1, 128), lambda i, t, k: (i, 0, k, 0, 0)),
            pl.BlockSpec((1, n, 3), lambda i, t, k: (i, 0, 0)),
            pl.BlockSpec((1, s_tile, 3), lambda i, t, k: (i, t, 0)),
            pl.BlockSpec((3, cdim), lambda i, t, k: (0, 0)),
            pl.BlockSpec((3, cdim), lambda i, t, k: (0, 0)),
            pl.BlockSpec((1, cdim), lambda i, t, k: (0, 0)),
        ],
        out_specs=pl.BlockSpec((1, s_tile, cdim), lambda i, t, k: (i, t, 0)),
        out_shape=jax.ShapeDtypeStruct((b, s_total, cdim), F32),
        scratch_shapes=[pltpu.VMEM((n, cdim), F32),
                        pltpu.VMEM((s_tile, cdim), F32),
                        pltpu.VMEM((s_tile, 1), F32)],
    )(xyz4, xyzc, cenc, wsum, wa, bvec)


def _sa2(xyz4, xyzc, cenc, pts, wa, wp, bvec, *, nsample, r2, s_tile):
    b, n3, nchunk = xyz4.shape[:3]
    n = nchunk * 128
    s_total = cenc.shape[1]
    pdim = wp.shape[0]
    cdim = wa.shape[1]
    return pl.pallas_call(
        functools.partial(_sa2_body, nchunk=nchunk, nsample=nsample, r2=r2),
        grid=(b, s_total // s_tile, nchunk),
        in_specs=[
            pl.BlockSpec((1, 3, 1, 1, 128), lambda i, t, k: (i, 0, k, 0, 0)),
            pl.BlockSpec((1, n, 3), lambda i, t, k: (i, 0, 0)),
            pl.BlockSpec((1, s_tile, 3), lambda i, t, k: (i, t, 0)),
            pl.BlockSpec((1, n, pdim), lambda i, t, k: (i, 0, 0)),
            pl.BlockSpec((3, cdim), lambda i, t, k: (0, 0)),
            pl.BlockSpec((pdim, cdim), lambda i, t, k: (0, 0)),
            pl.BlockSpec((1, cdim), lambda i, t, k: (0, 0)),
        ],
        out_specs=pl.BlockSpec((1, s_tile, cdim), lambda i, t, k: (i, t, 0)),
        out_shape=jax.ShapeDtypeStruct((b, s_total, cdim), F32),
        scratch_shapes=[pltpu.VMEM((n, cdim), F32),
                        pltpu.VMEM((s_tile, cdim), F32),
                        pltpu.VMEM((s_tile, 1), F32)],
    )(xyz4, xyzc, cenc, pts, wa, wp, bvec)


def _sa3fp3_body(cenc_ref, pts_ref, w3a_ref, w3p_ref, b3_ref,
                 wf3p_ref, wf3i_ref, bf3_ref, l3_ref, out_ref):
    pts = pts_ref[0]
    feat = jnp.maximum(
        jnp.dot(cenc_ref[0], w3a_ref[:, :], preferred_element_type=F32)
        + jnp.dot(pts, w3p_ref[:, :], preferred_element_type=F32)
        + b3_ref[0:1, :], 0.0)
    l3 = jnp.max(feat, axis=0, keepdims=True)
    l3_ref[0] = l3
    interp = jnp.dot(l3, wf3i_ref[:, :], preferred_element_type=F32)
    out_ref[0] = jnp.maximum(
        jnp.dot(pts, wf3p_ref[:, :], preferred_element_type=F32)
        + interp + bf3_ref[0:1, :], 0.0)


def _sa3fp3(cenc2, l2p, w3a, w3p, b3, wf3p, wf3i, bf3):
    b, s, _ = cenc2.shape
    pdim = l2p.shape[2]
    c3 = w3a.shape[1]
    cf = wf3p.shape[1]
    return pl.pallas_call(
        _sa3fp3_body,
        grid=(b,),
        in_specs=[
            pl.BlockSpec((1, s, 3), lambda i: (i, 0, 0)),
            pl.BlockSpec((1, s, pdim), lambda i: (i, 0, 0)),
            pl.BlockSpec((3, c3), lambda i: (0, 0)),
            pl.BlockSpec((pdim, c3), lambda i: (0, 0)),
            pl.BlockSpec((1, c3), lambda i: (0, 0)),
            pl.BlockSpec((pdim, cf), lambda i: (0, 0)),
            pl.BlockSpec((c3, cf), lambda i: (0, 0)),
            pl.BlockSpec((1, cf), lambda i: (0, 0)),
        ],
        out_specs=[
            pl.BlockSpec((1, 1, c3), lambda i: (i, 0, 0)),
            pl.BlockSpec((1, s, cf), lambda i: (i, 0, 0)),
        ],
        out_shape=[
            jax.ShapeDtypeStruct((b, 1, c3), F32),
            jax.ShapeDtypeStruct((b, s, cf), F32),
        ],
    )(cenc2, l2p, w3a, w3p, b3, wf3p, wf3i, bf3)


def _knn3_interp(d, pts):
    r, s2 = d.shape
    iota = jax.lax.broadcasted_iota(jnp.int32, (r, s2), 1)
    dd = d
    ohs = []
    ms = []
    for _ in range(3):
        m = jnp.min(dd, axis=1, keepdims=True)
        ii = jnp.min(jnp.where(dd == m, iota, s2), axis=1, keepdims=True)
        oh = iota == ii
        ms.append(m)
        ohs.append(oh)
        dd = jnp.where(oh, 3e38, dd)
    r1 = 1.0 / (ms[0] + 1e-8)
    r2 = 1.0 / (ms[1] + 1e-8)
    r3 = 1.0 / (ms[2] + 1e-8)
    norm = r1 + r2 + r3
    wh = (jnp.where(ohs[0], r1 / norm, 0.0)
          + jnp.where(ohs[1], r2 / norm, 0.0)
          + jnp.where(ohs[2], r3 / norm, 0.0))
    return jnp.dot(wh, pts, preferred_element_type=F32)


def _fp2_body(cen2_ref, cenc1_ref, l1p_ref, l2pn_ref, wp_ref, wi_ref,
              b_ref, out_ref):
    c1 = cenc1_ref[0]
    x2 = cen2_ref[0, 0:1, :]
    y2 = cen2_ref[0, 1:2, :]
    z2 = cen2_ref[0, 2:3, :]
    d = ((c1[:, 0:1] - x2) ** 2 + (c1[:, 1:2] - y2) ** 2
         + (c1[:, 2:3] - z2) ** 2)
    interp = _knn3_interp(d, l2pn_ref[0])
    out_ref[0] = jnp.maximum(
        jnp.dot(l1p_ref[0], wp_ref[:, :], preferred_element_type=F32)
        + jnp.dot(interp, wi_ref[:, :], preferred_element_type=F32)
        + b_ref[0:1, :], 0.0)


def _fp2(cen2, cenc1, l1p, l2pn, wp, wi, bvec):
    b, _, s2 = cen2.shape
    n1 = cenc1.shape[1]
    pdim = wp.shape[0]
    idim = wi.shape[0]
    cdim = wp.shape[1]
    return pl.pallas_call(
        _fp2_body,
        grid=(b,),
        in_specs=[
            pl.BlockSpec((1, 3, s2), lambda i: (i, 0, 0)),
            pl.BlockSpec((1, n1, 3), lambda i: (i, 0, 0)),
            pl.BlockSpec((1, n1, pdim), lambda i: (i, 0, 0)),
            pl.BlockSpec((1, s2, idim), lambda i: (i, 0, 0)),
            pl.BlockSpec((pdim, cdim), lambda i: (0, 0)),
            pl.BlockSpec((idim, cdim), lambda i: (0, 0)),
            pl.BlockSpec((1, cdim), lambda i: (0, 0)),
        ],
        out_specs=pl.BlockSpec((1, n1, cdim), lambda i: (i, 0, 0)),
        out_shape=jax.ShapeDtypeStruct((b, n1, cdim), F32),
    )(cen2, cenc1, l1p, l2pn, wp, wi, bvec)


def _fp1_body(cen1_ref, xyzc_ref, cls_ref, l1pn_ref, wc16_ref, wx_ref,
              wi_ref, b_ref, wcls_ref, bcls_ref, out_ref):
    c0 = xyzc_ref[0]
    x1 = cen1_ref[0, 0:1, :]
    y1 = cen1_ref[0, 1:2, :]
    z1 = cen1_ref[0, 2:3, :]
    d = ((c0[:, 0:1] - x1) ** 2 + (c0[:, 1:2] - y1) ** 2
         + (c0[:, 2:3] - z1) ** 2)
    interp = _knn3_interp(d, l1pn_ref[0])
    cls_part = jnp.dot(cls_ref[0], wc16_ref[:, :],
                       preferred_element_type=F32)
    feat = jnp.maximum(
        jnp.dot(interp, wi_ref[:, :], preferred_element_type=F32)
        + jnp.dot(c0, wx_ref[:, :], preferred_element_type=F32)
        + cls_part + b_ref[0:1, :], 0.0)
    logits = (jnp.dot(feat, wcls_ref[:, :], preferred_element_type=F32)
              + bcls_ref[0:1, :])
    m = jnp.max(logits, axis=1, keepdims=True)
    sh = logits - m
    out_ref[0] = sh - jnp.log(jnp.sum(jnp.exp(sh), axis=1, keepdims=True))


def _fp1(cen1, xyzc, cls3, l1pn, wc16, wx, wi, bvec, wcls, bcls, *, r_tile):
    b, _, s1 = cen1.shape
    n = xyzc.shape[1]
    idim = wi.shape[0]
    cdim = wi.shape[1]
    ncls = wcls.shape[1]
    return pl.pallas_call(
        _fp1_body,
        grid=(b, n // r_tile),
        in_specs=[
            pl.BlockSpec((1, 3, s1), lambda i, t: (i, 0, 0)),
            pl.BlockSpec((1, r_tile, 3), lambda i, t: (i, t, 0)),
            pl.BlockSpec((1, 1, 16), lambda i, t: (i, 0, 0)),
            pl.BlockSpec((1, s1, idim), lambda i, t: (i, 0, 0)),
            pl.BlockSpec((16, cdim), lambda i, t: (0, 0)),
            pl.BlockSpec((3, cdim), lambda i, t: (0, 0)),
            pl.BlockSpec((idim, cdim), lambda i, t: (0, 0)),
            pl.BlockSpec((1, cdim), lambda i, t: (0, 0)),
            pl.BlockSpec((cdim, ncls), lambda i, t: (0, 0)),
            pl.BlockSpec((1, ncls), lambda i, t: (0, 0)),
        ],
        out_specs=pl.BlockSpec((1, r_tile, ncls), lambda i, t: (i, t, 0)),
        out_shape=jax.ShapeDtypeStruct((b, n, ncls), F32),
    )(cen1, xyzc, cls3, l1pn, wc16, wx, wi, bvec, wcls, bcls)


def kernel(xyz, cls_label, W1, b1, W2, b2, W3, b3, Wf3, bf3, Wf2, bf2,
           Wf1, bf1, Wc, bc):
    b, _, n = xyz.shape
    xyzc = jnp.transpose(xyz, (0, 2, 1))
    xyz4 = xyz.reshape(b, 3, n // 128, 1, 128)

    cenc1 = _fps(xyz, 512)
    cen1 = jnp.transpose(cenc1, (0, 2, 1))

    l1p = _sa1(xyz4, xyzc, cenc1,
               W1[0:3] + W1[3:6], W1[0:3], b1.reshape(1, -1),
               nsample=32, r2=float(0.1 ** 2), s_tile=64)

    cenc2 = _fps(cen1, 128)
    cen2 = jnp.transpose(cenc2, (0, 2, 1))

    l2p = _sa2(cen1.reshape(b, 3, 4, 1, 128), cenc1, cenc2, l1p,
               W2[0:3], W2[3:], b2.reshape(1, -1),
               nsample=64, r2=float(0.3 ** 2), s_tile=64)

    l3, l2pn = _sa3fp3(cenc2, l2p, W3[0:3], W3[3:], b3.reshape(1, -1),
                       Wf3[0:256], Wf3[256:], bf3.reshape(1, -1))

    l1pn = _fp2(cen2, cenc1, l1p, l2pn,
                Wf2[0:128], Wf2[128:], bf2.reshape(1, -1))

    out = _fp1(cen1, xyzc, cls_label.reshape(b, 1, 16), l1pn,
               Wf1[0:16], Wf1[16:19] + Wf1[19:22], Wf1[22:],
               bf1.reshape(1, -1), Wc, bc.reshape(1, -1), r_tile=128)

    return out, jnp.transpose(l3, (0, 2, 1))


# speedup vs baseline: 6.2483x; 1.5496x over previous
import functools

import jax
import jax.numpy as jnp
from jax.experimental import pallas as pl
from jax.experimental.pallas import tpu as pltpu

F32 = jnp.float32
NEG = -1e30


def _fps_body(xyz_ref, out_ref, dist_ref, *, npoint, n):
    x = xyz_ref[:, 0, :]
    y = xyz_ref[:, 1, :]
    z = xyz_ref[:, 2, :]
    b = x.shape[0]
    dist_ref[:, :] = jnp.full((b, n), 1e10, F32)
    iota = jax.lax.broadcasted_iota(jnp.int32, (b, n), 1)

    def step(i, carry):
        cx, cy, cz = carry
        out_ref[:, pl.ds(i, 1), 0:1] = cx[:, :, None]
        out_ref[:, pl.ds(i, 1), 1:2] = cy[:, :, None]
        out_ref[:, pl.ds(i, 1), 2:3] = cz[:, :, None]
        d = (x - cx) ** 2 + (y - cy) ** 2 + (z - cz) ** 2
        dist = jnp.minimum(dist_ref[:, :], d)
        dist_ref[:, :] = dist
        m = jnp.max(dist, axis=1, keepdims=True)
        ii = jnp.min(jnp.where(dist == m, iota, n), axis=1, keepdims=True)
        oh = iota == ii
        ncx = jnp.sum(jnp.where(oh, x, 0.0), axis=1, keepdims=True)
        ncy = jnp.sum(jnp.where(oh, y, 0.0), axis=1, keepdims=True)
        ncz = jnp.sum(jnp.where(oh, z, 0.0), axis=1, keepdims=True)
        return ncx, ncy, ncz

    c0 = (x[:, 0:1], y[:, 0:1], z[:, 0:1])
    jax.lax.fori_loop(0, npoint, step, c0)


def _fps(xyz_planes, npoint):
    b, _, n = xyz_planes.shape
    return pl.pallas_call(
        functools.partial(_fps_body, npoint=npoint, n=n),
        out_shape=jax.ShapeDtypeStruct((b, npoint, 3), F32),
        scratch_shapes=[pltpu.VMEM((b, n), F32)],
    )(xyz_planes)


def _feat1_body(xyzc_ref, wsum_ref, out_ref):
    out_ref[0] = jnp.dot(xyzc_ref[0], wsum_ref[:, :],
                         preferred_element_type=F32)


def _feat2_body(xyzc_ref, pts_ref, wa_ref, wp_ref, out_ref):
    out_ref[0] = (
        jnp.dot(xyzc_ref[0], wa_ref[:, :], preferred_element_type=F32)
        + jnp.dot(pts_ref[0], wp_ref[:, :], preferred_element_type=F32))


def _feat1(xyzc, wsum):
    b, n, _ = xyzc.shape
    cdim = wsum.shape[1]
    return pl.pallas_call(
        _feat1_body,
        grid=(b,),
        in_specs=[
            pl.BlockSpec((1, n, 3), lambda i: (i, 0, 0)),
            pl.BlockSpec((3, cdim), lambda i: (0, 0)),
        ],
        out_specs=pl.BlockSpec((1, n, cdim), lambda i: (i, 0, 0)),
        out_shape=jax.ShapeDtypeStruct((b, n, cdim), F32),
    )(xyzc, wsum)


def _feat2(xyzc, pts, wa, wp):
    b, n, _ = xyzc.shape
    pdim = wp.shape[0]
    cdim = wa.shape[1]
    return pl.pallas_call(
        _feat2_body,
        grid=(b,),
        in_specs=[
            pl.BlockSpec((1, n, 3), lambda i: (i, 0, 0)),
            pl.BlockSpec((1, n, pdim), lambda i: (i, 0, 0)),
            pl.BlockSpec((3, cdim), lambda i: (0, 0)),
            pl.BlockSpec((pdim, cdim), lambda i: (0, 0)),
        ],
        out_specs=pl.BlockSpec((1, n, cdim), lambda i: (i, 0, 0)),
        out_shape=jax.ShapeDtypeStruct((b, n, cdim), F32),
    )(xyzc, pts, wa, wp)


def _ballidx_body(xyz4_ref, cenc_ref, out_ref, idxc_ref, off_ref,
                  *, nchunk, nsample, r2):
    k = pl.program_id(2)
    cen = cenc_ref[0]
    s = cen.shape[0]

    @pl.when(k == 0)
    def _():
        idxc_ref[:, :] = jnp.zeros((s, nsample), F32)
        off_ref[:, :] = jnp.zeros((s, 1), F32)

    cx = cen[:, 0:1]
    cy = cen[:, 1:2]
    cz = cen[:, 2:3]
    rit = jax.lax.broadcasted_iota(jnp.int32, (128, 128), 0)
    cit = jax.lax.broadcasted_iota(jnp.int32, (128, 128), 1)
    ut = (rit <= cit).astype(F32)

    x = xyz4_ref[0, 0, 0, 0:1, :]
    y = xyz4_ref[0, 1, 0, 0:1, :]
    z = xyz4_ref[0, 2, 0, 0:1, :]
    d = (cx - x) ** 2 + (cy - y) ** 2 + (cz - z) ** 2
    inb = d <= r2
    pc = (jnp.dot(inb.astype(F32), ut, preferred_element_type=F32)
          + off_ref[:, :])
    sel = jnp.logical_and(inb, pc <= float(nsample))
    jv = (jax.lax.broadcasted_iota(jnp.int32, (1, 128), 1)
          + k * 128 + 1).astype(F32)
    v = jnp.where(sel, jv, 0.0)
    for q in range(nsample):
        mq = jnp.where(pc == float(q + 1), v, 0.0)
        idxc_ref[:, q:q + 1] = (idxc_ref[:, q:q + 1]
                                + jnp.sum(mq, axis=1, keepdims=True))
    off_ref[:, :] = pc[:, 127:128]

    @pl.when(k == nchunk - 1)
    def _():
        out_ref[0] = idxc_ref[:, :] - 1.0

    return


def _ballidx(xyz4, cenc, *, nsample, r2, s_tile):
    b, n3, nchunk = xyz4.shape[:3]
    s_total = cenc.shape[1]
    return pl.pallas_call(
        functools.partial(_ballidx_body, nchunk=nchunk, nsample=nsample,
                          r2=r2),
        grid=(b, s_total // s_tile, nchunk),
        in_specs=[
            pl.BlockSpec((1, 3, 1, 1, 128), lambda i, t, k: (i, 0, k, 0, 0)),
            pl.BlockSpec((1, s_tile, 3), lambda i, t, k: (i, t, 0)),
        ],
        out_specs=pl.BlockSpec((1, s_tile, nsample),
                               lambda i, t, k: (i, t, 0)),
        out_shape=jax.ShapeDtypeStruct((b, s_total, nsample), F32),
        scratch_shapes=[pltpu.VMEM((s_tile, nsample), F32),
                        pltpu.VMEM((s_tile, 1), F32)],
    )(xyz4, cenc)


def _gmax_body(idx_ref, f_ref, cenc_ref, wa_ref, b_ref, out_ref,
               *, s_tile, nsample):
    rows = []
    for s in range(s_tile):
        i0 = idx_ref[0, s, 0]
        row = None
        for q in range(nsample):
            iv = idx_ref[0, s, q]
            ivs = jnp.where(iv < 0, i0, iv)
            fr = f_ref[0, pl.ds(ivs, 1), :]
            row = fr if row is None else jnp.maximum(row, fr)
        rows.append(row)
    acc = jnp.concatenate(rows, axis=0)
    cen = cenc_ref[0]
    a = jnp.dot(cen, wa_ref[:, :], preferred_element_type=F32)
    out_ref[0] = jnp.maximum(acc - a + b_ref[0:1, :], 0.0)


def _gmax(idx, feat, cenc, wa, bvec, *, s_tile):
    b, s_total, nsample = idx.shape
    n = feat.shape[1]
    cdim = feat.shape[2]
    return pl.pallas_call(
        functools.partial(_gmax_body, s_tile=s_tile, nsample=nsample),
        grid=(b, s_total // s_tile),
        in_specs=[
            pl.BlockSpec((1, s_tile, nsample), lambda i, t: (i, t, 0),
                         memory_space=pltpu.SMEM),
            pl.BlockSpec((1, n, cdim), lambda i, t: (i, 0, 0)),
            pl.BlockSpec((1, s_tile, 3), lambda i, t: (i, t, 0)),
            pl.BlockSpec((3, cdim), lambda i, t: (0, 0)),
            pl.BlockSpec((1, cdim), lambda i, t: (0, 0)),
        ],
        out_specs=pl.BlockSpec((1, s_tile, cdim), lambda i, t: (i, t, 0)),
        out_shape=jax.ShapeDtypeStruct((b, s_total, cdim), F32),
    )(idx, feat, cenc, wa, bvec)


def _sa1(xyz4, xyzc, cenc, wsum, wa, bvec, *, nsample, r2, s_tile):
    feat = _feat1(xyzc, wsum)
    idx = _ballidx(xyz4, cenc, nsample=nsample, r2=r2, s_tile=s_tile)
    return _gmax(idx.astype(jnp.int32), feat, cenc, wa, bvec, s_tile=32)


def _sa2(xyz4, xyzc, cenc, pts, wa, wp, bvec, *, nsample, r2, s_tile):
    feat = _feat2(xyzc, pts, wa, wp)
    idx = _ballidx(xyz4, cenc, nsample=nsample, r2=r2, s_tile=s_tile)
    return _gmax(idx.astype(jnp.int32), feat, cenc, wa, bvec, s_tile=32)


def _sa3fp3_body(cenc_ref, pts_ref, w3a_ref, w3p_ref, b3_ref,
                 wf3p_ref, wf3i_ref, bf3_ref, l3_ref, out_ref):
    pts = pts_ref[0]
    feat = jnp.maximum(
        jnp.dot(cenc_ref[0], w3a_ref[:, :], preferred_element_type=F32)
        + jnp.dot(pts, w3p_ref[:, :], preferred_element_type=F32)
        + b3_ref[0:1, :], 0.0)
    l3 = jnp.max(feat, axis=0, keepdims=True)
    l3_ref[0] = l3
    interp = jnp.dot(l3, wf3i_ref[:, :], preferred_element_type=F32)
    out_ref[0] = jnp.maximum(
        jnp.dot(pts, wf3p_ref[:, :], preferred_element_type=F32)
        + interp + bf3_ref[0:1, :], 0.0)


def _sa3fp3(cenc2, l2p, w3a, w3p, b3, wf3p, wf3i, bf3):
    b, s, _ = cenc2.shape
    pdim = l2p.shape[2]
    c3 = w3a.shape[1]
    cf = wf3p.shape[1]
    return pl.pallas_call(
        _sa3fp3_body,
        grid=(b,),
        in_specs=[
            pl.BlockSpec((1, s, 3), lambda i: (i, 0, 0)),
            pl.BlockSpec((1, s, pdim), lambda i: (i, 0, 0)),
            pl.BlockSpec((3, c3), lambda i: (0, 0)),
            pl.BlockSpec((pdim, c3), lambda i: (0, 0)),
            pl.BlockSpec((1, c3), lambda i: (0, 0)),
            pl.BlockSpec((pdim, cf), lambda i: (0, 0)),
            pl.BlockSpec((c3, cf), lambda i: (0, 0)),
            pl.BlockSpec((1, cf), lambda i: (0, 0)),
        ],
        out_specs=[
            pl.BlockSpec((1, 1, c3), lambda i: (i, 0, 0)),
            pl.BlockSpec((1, s, cf), lambda i: (i, 0, 0)),
        ],
        out_shape=[
            jax.ShapeDtypeStruct((b, 1, c3), F32),
            jax.ShapeDtypeStruct((b, s, cf), F32),
        ],
    )(cenc2, l2p, w3a, w3p, b3, wf3p, wf3i, bf3)


def _knn3_interp(d, pts):
    r, s2 = d.shape
    iota = jax.lax.broadcasted_iota(jnp.int32, (r, s2), 1)
    dd = d
    ohs = []
    ms = []
    for _ in range(3):
        m = jnp.min(dd, axis=1, keepdims=True)
        ii = jnp.min(jnp.where(dd == m, iota, s2), axis=1, keepdims=True)
        oh = iota == ii
        ms.append(m)
        ohs.append(oh)
        dd = jnp.where(oh, 3e38, dd)
    r1 = 1.0 / (ms[0] + 1e-8)
    r2 = 1.0 / (ms[1] + 1e-8)
    r3 = 1.0 / (ms[2] + 1e-8)
    norm = r1 + r2 + r3
    wh = (jnp.where(ohs[0], r1 / norm, 0.0)
          + jnp.where(ohs[1], r2 / norm, 0.0)
          + jnp.where(ohs[2], r3 / norm, 0.0))
    return jnp.dot(wh, pts, preferred_element_type=F32)


def _fp2_body(cen2_ref, cenc1_ref, l1p_ref, l2pn_ref, wp_ref, wi_ref,
              b_ref, out_ref):
    c1 = cenc1_ref[0]
    x2 = cen2_ref[0, 0:1, :]
    y2 = cen2_ref[0, 1:2, :]
    z2 = cen2_ref[0, 2:3, :]
    d = ((c1[:, 0:1] - x2) ** 2 + (c1[:, 1:2] - y2) ** 2
         + (c1[:, 2:3] - z2) ** 2)
    interp = _knn3_interp(d, l2pn_ref[0])
    out_ref[0] = jnp.maximum(
        jnp.dot(l1p_ref[0], wp_ref[:, :], preferred_element_type=F32)
        + jnp.dot(interp, wi_ref[:, :], preferred_element_type=F32)
        + b_ref[0:1, :], 0.0)


def _fp2(cen2, cenc1, l1p, l2pn, wp, wi, bvec):
    b, _, s2 = cen2.shape
    n1 = cenc1.shape[1]
    pdim = wp.shape[0]
    idim = wi.shape[0]
    cdim = wp.shape[1]
    return pl.pallas_call(
        _fp2_body,
        grid=(b,),
        in_specs=[
            pl.BlockSpec((1, 3, s2), lambda i: (i, 0, 0)),
            pl.BlockSpec((1, n1, 3), lambda i: (i, 0, 0)),
            pl.BlockSpec((1, n1, pdim), lambda i: (i, 0, 0)),
            pl.BlockSpec((1, s2, idim), lambda i: (i, 0, 0)),
            pl.BlockSpec((pdim, cdim), lambda i: (0, 0)),
            pl.BlockSpec((idim, cdim), lambda i: (0, 0)),
            pl.BlockSpec((1, cdim), lambda i: (0, 0)),
        ],
        out_specs=pl.BlockSpec((1, n1, cdim), lambda i: (i, 0, 0)),
        out_shape=jax.ShapeDtypeStruct((b, n1, cdim), F32),
    )(cen2, cenc1, l1p, l2pn, wp, wi, bvec)


def _fp1_body(cen1_ref, xyzc_ref, cls_ref, l1pn_ref, wc16_ref, wx_ref,
              wi_ref, b_ref, wcls_ref, bcls_ref, out_ref):
    c0 = xyzc_ref[0]
    x1 = cen1_ref[0, 0:1, :]
    y1 = cen1_ref[0, 1:2, :]
    z1 = cen1_ref[0, 2:3, :]
    d = ((c0[:, 0:1] - x1) ** 2 + (c0[:, 1:2] - y1) ** 2
         + (c0[:, 2:3] - z1) ** 2)
    interp = _knn3_interp(d, l1pn_ref[0])
    cls_part = jnp.dot(cls_ref[0], wc16_ref[:, :],
                       preferred_element_type=F32)
    feat = jnp.maximum(
        jnp.dot(interp, wi_ref[:, :], preferred_element_type=F32)
        + jnp.dot(c0, wx_ref[:, :], preferred_element_type=F32)
        + cls_part + b_ref[0:1, :], 0.0)
    logits = (jnp.dot(feat, wcls_ref[:, :], preferred_element_type=F32)
              + bcls_ref[0:1, :])
    m = jnp.max(logits, axis=1, keepdims=True)
    sh = logits - m
    out_ref[0] = sh - jnp.log(jnp.sum(jnp.exp(sh), axis=1, keepdims=True))


def _fp1(cen1, xyzc, cls3, l1pn, wc16, wx, wi, bvec, wcls, bcls, *, r_tile):
    b, _, s1 = cen1.shape
    n = xyzc.shape[1]
    idim = wi.shape[0]
    cdim = wi.shape[1]
    ncls = wcls.shape[1]
    return pl.pallas_call(
        _fp1_body,
        grid=(b, n // r_tile),
        in_specs=[
            pl.BlockSpec((1, 3, s1), lambda i, t: (i, 0, 0)),
            pl.BlockSpec((1, r_tile, 3), lambda i, t: (i, t, 0)),
            pl.BlockSpec((1, 1, 16), lambda i, t: (i, 0, 0)),
            pl.BlockSpec((1, s1, idim), lambda i, t: (i, 0, 0)),
            pl.BlockSpec((16, cdim), lambda i, t: (0, 0)),
            pl.BlockSpec((3, cdim), lambda i, t: (0, 0)),
            pl.BlockSpec((idim, cdim), lambda i, t: (0, 0)),
            pl.BlockSpec((1, cdim), lambda i, t: (0, 0)),
            pl.BlockSpec((cdim, ncls), lambda i, t: (0, 0)),
            pl.BlockSpec((1, ncls), lambda i, t: (0, 0)),
        ],
        out_specs=pl.BlockSpec((1, r_tile, ncls), lambda i, t: (i, t, 0)),
        out_shape=jax.ShapeDtypeStruct((b, n, ncls), F32),
    )(cen1, xyzc, cls3, l1pn, wc16, wx, wi, bvec, wcls, bcls)


def kernel(xyz, cls_label, W1, b1, W2, b2, W3, b3, Wf3, bf3, Wf2, bf2,
           Wf1, bf1, Wc, bc):
    b, _, n = xyz.shape
    xyzc = jnp.transpose(xyz, (0, 2, 1))
    xyz4 = xyz.reshape(b, 3, n // 128, 1, 128)

    cenc1 = _fps(xyz, 512)
    cen1 = jnp.transpose(cenc1, (0, 2, 1))

    l1p = _sa1(xyz4, xyzc, cenc1,
               W1[0:3] + W1[3:6], W1[0:3], b1.reshape(1, -1),
               nsample=32, r2=float(0.1 ** 2), s_tile=64)

    cenc2 = _fps(cen1, 128)
    cen2 = jnp.transpose(cenc2, (0, 2, 1))

    l2p = _sa2(cen1.reshape(b, 3, 4, 1, 128), cenc1, cenc2, l1p,
               W2[0:3], W2[3:], b2.reshape(1, -1),
               nsample=64, r2=float(0.3 ** 2), s_tile=64)

    l3, l2pn = _sa3fp3(cenc2, l2p, W3[0:3], W3[3:], b3.reshape(1, -1),
                       Wf3[0:256], Wf3[256:], bf3.reshape(1, -1))

    l1pn = _fp2(cen2, cenc1, l1p, l2pn,
                Wf2[0:128], Wf2[128:], bf2.reshape(1, -1))

    out = _fp1(cen1, xyzc, cls_label.reshape(b, 1, 16), l1pn,
               Wf1[0:16], Wf1[16:19] + Wf1[19:22], Wf1[22:],
               bf1.reshape(1, -1), Wc, bc.reshape(1, -1), r_tile=128)

    return out, jnp.transpose(l3, (0, 2, 1))


# speedup vs baseline: 7.8602x; 1.2580x over previous
import functools

import jax
import jax.numpy as jnp
from jax.experimental import pallas as pl
from jax.experimental.pallas import tpu as pltpu

F32 = jnp.float32
NEG = -1e30


def _fps_body(xyz_ref, out_ref, dist_ref, *, npoint, n):
    x = xyz_ref[:, 0, :]
    y = xyz_ref[:, 1, :]
    z = xyz_ref[:, 2, :]
    b = x.shape[0]
    dist_ref[:, :] = jnp.full((b, n), 1e10, F32)
    iota = jax.lax.broadcasted_iota(jnp.int32, (b, n), 1)

    def step(i, carry):
        cx, cy, cz = carry
        out_ref[:, pl.ds(i, 1), 0:1] = cx[:, :, None]
        out_ref[:, pl.ds(i, 1), 1:2] = cy[:, :, None]
        out_ref[:, pl.ds(i, 1), 2:3] = cz[:, :, None]
        d = (x - cx) ** 2 + (y - cy) ** 2 + (z - cz) ** 2
        dist = jnp.minimum(dist_ref[:, :], d)
        dist_ref[:, :] = dist
        m = jnp.max(dist, axis=1, keepdims=True)
        ii = jnp.min(jnp.where(dist == m, iota, n), axis=1, keepdims=True)
        oh = iota == ii
        ncx = jnp.sum(jnp.where(oh, x, 0.0), axis=1, keepdims=True)
        ncy = jnp.sum(jnp.where(oh, y, 0.0), axis=1, keepdims=True)
        ncz = jnp.sum(jnp.where(oh, z, 0.0), axis=1, keepdims=True)
        return ncx, ncy, ncz

    c0 = (x[:, 0:1], y[:, 0:1], z[:, 0:1])
    jax.lax.fori_loop(0, npoint, step, c0)


def _fps(xyz_planes, npoint):
    b, _, n = xyz_planes.shape
    return pl.pallas_call(
        functools.partial(_fps_body, npoint=npoint, n=n),
        out_shape=jax.ShapeDtypeStruct((b, npoint, 3), F32),
        scratch_shapes=[pltpu.VMEM((b, n), F32)],
    )(xyz_planes)


def _feat1_body(xyzc_ref, wsum_ref, out_ref):
    out_ref[0] = jnp.dot(xyzc_ref[0], wsum_ref[:, :],
                         preferred_element_type=F32)


def _feat2_body(xyzc_ref, pts_ref, wa_ref, wp_ref, out_ref):
    out_ref[0] = (
        jnp.dot(xyzc_ref[0], wa_ref[:, :], preferred_element_type=F32)
        + jnp.dot(pts_ref[0], wp_ref[:, :], preferred_element_type=F32))


def _feat1(xyzc, wsum):
    b, n, _ = xyzc.shape
    cdim = wsum.shape[1]
    return pl.pallas_call(
        _feat1_body,
        grid=(b,),
        in_specs=[
            pl.BlockSpec((1, n, 3), lambda i: (i, 0, 0)),
            pl.BlockSpec((3, cdim), lambda i: (0, 0)),
        ],
        out_specs=pl.BlockSpec((1, n, cdim), lambda i: (i, 0, 0)),
        out_shape=jax.ShapeDtypeStruct((b, n, cdim), F32),
    )(xyzc, wsum)


def _feat2(xyzc, pts, wa, wp):
    b, n, _ = xyzc.shape
    pdim = wp.shape[0]
    cdim = wa.shape[1]
    return pl.pallas_call(
        _feat2_body,
        grid=(b,),
        in_specs=[
            pl.BlockSpec((1, n, 3), lambda i: (i, 0, 0)),
            pl.BlockSpec((1, n, pdim), lambda i: (i, 0, 0)),
            pl.BlockSpec((3, cdim), lambda i: (0, 0)),
            pl.BlockSpec((pdim, cdim), lambda i: (0, 0)),
        ],
        out_specs=pl.BlockSpec((1, n, cdim), lambda i: (i, 0, 0)),
        out_shape=jax.ShapeDtypeStruct((b, n, cdim), F32),
    )(xyzc, pts, wa, wp)


def _ballidx_body(xyz4_ref, cenc_ref, out_ref, idxc_ref, off_ref,
                  *, nchunk, nsample, r2):
    k = pl.program_id(2)
    cen = cenc_ref[0]
    s = cen.shape[0]

    @pl.when(k == 0)
    def _():
        idxc_ref[:, :] = jnp.zeros((s, nsample), F32)
        off_ref[:, :] = jnp.zeros((s, 1), F32)

    cx = cen[:, 0:1]
    cy = cen[:, 1:2]
    cz = cen[:, 2:3]
    rit = jax.lax.broadcasted_iota(jnp.int32, (128, 128), 0)
    cit = jax.lax.broadcasted_iota(jnp.int32, (128, 128), 1)
    ut = (rit <= cit).astype(F32)

    x = xyz4_ref[0, 0, 0, 0:1, :]
    y = xyz4_ref[0, 1, 0, 0:1, :]
    z = xyz4_ref[0, 2, 0, 0:1, :]
    d = (cx - x) ** 2 + (cy - y) ** 2 + (cz - z) ** 2
    inb = d <= r2
    pc = (jnp.dot(inb.astype(F32), ut, preferred_element_type=F32)
          + off_ref[:, :])
    sel = jnp.logical_and(inb, pc <= float(nsample))
    jv = (jax.lax.broadcasted_iota(jnp.int32, (1, 128), 1)
          + k * 128 + 1).astype(F32)
    v = jnp.where(sel, jv, 0.0)
    cols = [jnp.sum(jnp.where(pc == float(q + 1), v, 0.0),
                    axis=1, keepdims=True)
            for q in range(nsample)]
    idxc_ref[:, :] = idxc_ref[:, :] + jnp.concatenate(cols, axis=1)
    off_ref[:, :] = pc[:, 127:128]

    @pl.when(k == nchunk - 1)
    def _():
        out_ref[0] = idxc_ref[:, :] - 1.0

    return


def _ballidx(xyz4, cenc, *, nsample, r2, s_tile):
    b, n3, nchunk = xyz4.shape[:3]
    s_total = cenc.shape[1]
    return pl.pallas_call(
        functools.partial(_ballidx_body, nchunk=nchunk, nsample=nsample,
                          r2=r2),
        grid=(b, s_total // s_tile, nchunk),
        in_specs=[
            pl.BlockSpec((1, 3, 1, 1, 128), lambda i, t, k: (i, 0, k, 0, 0)),
            pl.BlockSpec((1, s_tile, 3), lambda i, t, k: (i, t, 0)),
        ],
        out_specs=pl.BlockSpec((1, s_tile, nsample),
                               lambda i, t, k: (i, t, 0)),
        out_shape=jax.ShapeDtypeStruct((b, s_total, nsample), F32),
        scratch_shapes=[pltpu.VMEM((s_tile, nsample), F32),
                        pltpu.VMEM((s_tile, 1), F32)],
    )(xyz4, cenc)


def _gmax_body(idx_ref, f_ref, cenc_ref, wa_ref, b_ref, out_ref,
               *, s_tile, nsample):
    rows = []
    for s in range(s_tile):
        i0 = idx_ref[0, s, 0]
        row = None
        for q in range(nsample):
            iv = idx_ref[0, s, q]
            ivs = jnp.where(iv < 0, i0, iv)
            fr = f_ref[0, pl.ds(ivs, 1), :]
            row = fr if row is None else jnp.maximum(row, fr)
        rows.append(row)
    acc = jnp.concatenate(rows, axis=0)
    cen = cenc_ref[0]
    a = jnp.dot(cen, wa_ref[:, :], preferred_element_type=F32)
    out_ref[0] = jnp.maximum(acc - a + b_ref[0:1, :], 0.0)


def _gmax(idx, feat, cenc, wa, bvec, *, s_tile):
    b, s_total, nsample = idx.shape
    n = feat.shape[1]
    cdim = feat.shape[2]
    return pl.pallas_call(
        functools.partial(_gmax_body, s_tile=s_tile, nsample=nsample),
        grid=(b, s_total // s_tile),
        in_specs=[
            pl.BlockSpec((1, s_tile, nsample), lambda i, t: (i, t, 0),
                         memory_space=pltpu.SMEM),
            pl.BlockSpec((1, n, cdim), lambda i, t: (i, 0, 0)),
            pl.BlockSpec((1, s_tile, 3), lambda i, t: (i, t, 0)),
            pl.BlockSpec((3, cdim), lambda i, t: (0, 0)),
            pl.BlockSpec((1, cdim), lambda i, t: (0, 0)),
        ],
        out_specs=pl.BlockSpec((1, s_tile, cdim), lambda i, t: (i, t, 0)),
        out_shape=jax.ShapeDtypeStruct((b, s_total, cdim), F32),
    )(idx, feat, cenc, wa, bvec)


def _sa1(xyz4, xyzc, cenc, wsum, wa, bvec, *, nsample, r2, s_tile):
    feat = _feat1(xyzc, wsum)
    idx = _ballidx(xyz4, cenc, nsample=nsample, r2=r2, s_tile=128)
    return _gmax(idx.astype(jnp.int32), feat, cenc, wa, bvec, s_tile=32)


def _sa2(xyz4, xyzc, cenc, pts, wa, wp, bvec, *, nsample, r2, s_tile):
    feat = _feat2(xyzc, pts, wa, wp)
    idx = _ballidx(xyz4, cenc, nsample=nsample, r2=r2, s_tile=s_tile)
    return _gmax(idx.astype(jnp.int32), feat, cenc, wa, bvec, s_tile=32)


def _sa3fp3_body(cenc_ref, pts_ref, w3a_ref, w3p_ref, b3_ref,
                 wf3p_ref, wf3i_ref, bf3_ref, l3_ref, out_ref):
    pts = pts_ref[0]
    feat = jnp.maximum(
        jnp.dot(cenc_ref[0], w3a_ref[:, :], preferred_element_type=F32)
        + jnp.dot(pts, w3p_ref[:, :], preferred_element_type=F32)
        + b3_ref[0:1, :], 0.0)
    l3 = jnp.max(feat, axis=0, keepdims=True)
    l3_ref[0] = l3
    interp = jnp.dot(l3, wf3i_ref[:, :], preferred_element_type=F32)
    out_ref[0] = jnp.maximum(
        jnp.dot(pts, wf3p_ref[:, :], preferred_element_type=F32)
        + interp + bf3_ref[0:1, :], 0.0)


def _sa3fp3(cenc2, l2p, w3a, w3p, b3, wf3p, wf3i, bf3):
    b, s, _ = cenc2.shape
    pdim = l2p.shape[2]
    c3 = w3a.shape[1]
    cf = wf3p.shape[1]
    return pl.pallas_call(
        _sa3fp3_body,
        grid=(b,),
        in_specs=[
            pl.BlockSpec((1, s, 3), lambda i: (i, 0, 0)),
            pl.BlockSpec((1, s, pdim), lambda i: (i, 0, 0)),
            pl.BlockSpec((3, c3), lambda i: (0, 0)),
            pl.BlockSpec((pdim, c3), lambda i: (0, 0)),
            pl.BlockSpec((1, c3), lambda i: (0, 0)),
            pl.BlockSpec((pdim, cf), lambda i: (0, 0)),
            pl.BlockSpec((c3, cf), lambda i: (0, 0)),
            pl.BlockSpec((1, cf), lambda i: (0, 0)),
        ],
        out_specs=[
            pl.BlockSpec((1, 1, c3), lambda i: (i, 0, 0)),
            pl.BlockSpec((1, s, cf), lambda i: (i, 0, 0)),
        ],
        out_shape=[
            jax.ShapeDtypeStruct((b, 1, c3), F32),
            jax.ShapeDtypeStruct((b, s, cf), F32),
        ],
    )(cenc2, l2p, w3a, w3p, b3, wf3p, wf3i, bf3)


def _knn3_interp(d, pts):
    r, s2 = d.shape
    iota = jax.lax.broadcasted_iota(jnp.int32, (r, s2), 1)
    dd = d
    ohs = []
    ms = []
    for _ in range(3):
        m = jnp.min(dd, axis=1, keepdims=True)
        ii = jnp.min(jnp.where(dd == m, iota, s2), axis=1, keepdims=True)
        oh = iota == ii
        ms.append(m)
        ohs.append(oh)
        dd = jnp.where(oh, 3e38, dd)
    r1 = 1.0 / (ms[0] + 1e-8)
    r2 = 1.0 / (ms[1] + 1e-8)
    r3 = 1.0 / (ms[2] + 1e-8)
    norm = r1 + r2 + r3
    wh = (jnp.where(ohs[0], r1 / norm, 0.0)
          + jnp.where(ohs[1], r2 / norm, 0.0)
          + jnp.where(ohs[2], r3 / norm, 0.0))
    return jnp.dot(wh, pts, preferred_element_type=F32)


def _fp2_body(cen2_ref, cenc1_ref, l1p_ref, l2pn_ref, wp_ref, wi_ref,
              b_ref, out_ref):
    c1 = cenc1_ref[0]
    x2 = cen2_ref[0, 0:1, :]
    y2 = cen2_ref[0, 1:2, :]
    z2 = cen2_ref[0, 2:3, :]
    d = ((c1[:, 0:1] - x2) ** 2 + (c1[:, 1:2] - y2) ** 2
         + (c1[:, 2:3] - z2) ** 2)
    interp = _knn3_interp(d, l2pn_ref[0])
    out_ref[0] = jnp.maximum(
        jnp.dot(l1p_ref[0], wp_ref[:, :], preferred_element_type=F32)
        + jnp.dot(interp, wi_ref[:, :], preferred_element_type=F32)
        + b_ref[0:1, :], 0.0)


def _fp2(cen2, cenc1, l1p, l2pn, wp, wi, bvec):
    b, _, s2 = cen2.shape
    n1 = cenc1.shape[1]
    pdim = wp.shape[0]
    idim = wi.shape[0]
    cdim = wp.shape[1]
    return pl.pallas_call(
        _fp2_body,
        grid=(b,),
        in_specs=[
            pl.BlockSpec((1, 3, s2), lambda i: (i, 0, 0)),
            pl.BlockSpec((1, n1, 3), lambda i: (i, 0, 0)),
            pl.BlockSpec((1, n1, pdim), lambda i: (i, 0, 0)),
            pl.BlockSpec((1, s2, idim), lambda i: (i, 0, 0)),
            pl.BlockSpec((pdim, cdim), lambda i: (0, 0)),
            pl.BlockSpec((idim, cdim), lambda i: (0, 0)),
            pl.BlockSpec((1, cdim), lambda i: (0, 0)),
        ],
        out_specs=pl.BlockSpec((1, n1, cdim), lambda i: (i, 0, 0)),
        out_shape=jax.ShapeDtypeStruct((b, n1, cdim), F32),
    )(cen2, cenc1, l1p, l2pn, wp, wi, bvec)


def _fp1_body(cen1_ref, xyzc_ref, cls_ref, l1pn_ref, wc16_ref, wx_ref,
              wi_ref, b_ref, wcls_ref, bcls_ref, out_ref):
    c0 = xyzc_ref[0]
    x1 = cen1_ref[0, 0:1, :]
    y1 = cen1_ref[0, 1:2, :]
    z1 = cen1_ref[0, 2:3, :]
    d = ((c0[:, 0:1] - x1) ** 2 + (c0[:, 1:2] - y1) ** 2
         + (c0[:, 2:3] - z1) ** 2)
    interp = _knn3_interp(d, l1pn_ref[0])
    cls_part = jnp.dot(cls_ref[0], wc16_ref[:, :],
                       preferred_element_type=F32)
    feat = jnp.maximum(
        jnp.dot(interp, wi_ref[:, :], preferred_element_type=F32)
        + jnp.dot(c0, wx_ref[:, :], preferred_element_type=F32)
        + cls_part + b_ref[0:1, :], 0.0)
    logits = (jnp.dot(feat, wcls_ref[:, :], preferred_element_type=F32)
              + bcls_ref[0:1, :])
    m = jnp.max(logits, axis=1, keepdims=True)
    sh = logits - m
    out_ref[0] = sh - jnp.log(jnp.sum(jnp.exp(sh), axis=1, keepdims=True))


def _fp1(cen1, xyzc, cls3, l1pn, wc16, wx, wi, bvec, wcls, bcls, *, r_tile):
    b, _, s1 = cen1.shape
    n = xyzc.shape[1]
    idim = wi.shape[0]
    cdim = wi.shape[1]
    ncls = wcls.shape[1]
    return pl.pallas_call(
        _fp1_body,
        grid=(b, n // r_tile),
        in_specs=[
            pl.BlockSpec((1, 3, s1), lambda i, t: (i, 0, 0)),
            pl.BlockSpec((1, r_tile, 3), lambda i, t: (i, t, 0)),
            pl.BlockSpec((1, 1, 16), lambda i, t: (i, 0, 0)),
            pl.BlockSpec((1, s1, idim), lambda i, t: (i, 0, 0)),
            pl.BlockSpec((16, cdim), lambda i, t: (0, 0)),
            pl.BlockSpec((3, cdim), lambda i, t: (0, 0)),
            pl.BlockSpec((idim, cdim), lambda i, t: (0, 0)),
            pl.BlockSpec((1, cdim), lambda i, t: (0, 0)),
            pl.BlockSpec((cdim, ncls), lambda i, t: (0, 0)),
            pl.BlockSpec((1, ncls), lambda i, t: (0, 0)),
        ],
        out_specs=pl.BlockSpec((1, r_tile, ncls), lambda i, t: (i, t, 0)),
        out_shape=jax.ShapeDtypeStruct((b, n, ncls), F32),
    )(cen1, xyzc, cls3, l1pn, wc16, wx, wi, bvec, wcls, bcls)


def kernel(xyz, cls_label, W1, b1, W2, b2, W3, b3, Wf3, bf3, Wf2, bf2,
           Wf1, bf1, Wc, bc):
    b, _, n = xyz.shape
    xyzc = jnp.transpose(xyz, (0, 2, 1))
    xyz4 = xyz.reshape(b, 3, n // 128, 1, 128)

    cenc1 = _fps(xyz, 512)
    cen1 = jnp.transpose(cenc1, (0, 2, 1))

    l1p = _sa1(xyz4, xyzc, cenc1,
               W1[0:3] + W1[3:6], W1[0:3], b1.reshape(1, -1),
               nsample=32, r2=float(0.1 ** 2), s_tile=64)

    cenc2 = _fps(cen1, 128)
    cen2 = jnp.transpose(cenc2, (0, 2, 1))

    l2p = _sa2(cen1.reshape(b, 3, 4, 1, 128), cenc1, cenc2, l1p,
               W2[0:3], W2[3:], b2.reshape(1, -1),
               nsample=64, r2=float(0.3 ** 2), s_tile=64)

    l3, l2pn = _sa3fp3(cenc2, l2p, W3[0:3], W3[3:], b3.reshape(1, -1),
                       Wf3[0:256], Wf3[256:], bf3.reshape(1, -1))

    l1pn = _fp2(cen2, cenc1, l1p, l2pn,
                Wf2[0:128], Wf2[128:], bf2.reshape(1, -1))

    out = _fp1(cen1, xyzc, cls_label.reshape(b, 1, 16), l1pn,
               Wf1[0:16], Wf1[16:19] + Wf1[19:22], Wf1[22:],
               bf1.reshape(1, -1), Wc, bc.reshape(1, -1), r_tile=128)

    return out, jnp.transpose(l3, (0, 2, 1))


# speedup vs baseline: 9.0894x; 1.1564x over previous
import functools

import jax
import jax.numpy as jnp
from jax.experimental import pallas as pl
from jax.experimental.pallas import tpu as pltpu

F32 = jnp.float32
NEG = -1e30


def _fps_body(xyz_ref, out_ref, dist_ref, *, npoint, n):
    x = xyz_ref[:, 0, :]
    y = xyz_ref[:, 1, :]
    z = xyz_ref[:, 2, :]
    b = x.shape[0]
    dist_ref[:, :] = jnp.full((b, n), 1e10, F32)
    iota = jax.lax.broadcasted_iota(jnp.int32, (b, n), 1)

    def step(i, carry):
        cx, cy, cz = carry
        out_ref[:, pl.ds(i, 1), 0:1] = cx[:, :, None]
        out_ref[:, pl.ds(i, 1), 1:2] = cy[:, :, None]
        out_ref[:, pl.ds(i, 1), 2:3] = cz[:, :, None]
        d = (x - cx) ** 2 + (y - cy) ** 2 + (z - cz) ** 2
        dist = jnp.minimum(dist_ref[:, :], d)
        dist_ref[:, :] = dist
        m = jnp.max(dist, axis=1, keepdims=True)
        ii = jnp.min(jnp.where(dist == m, iota, n), axis=1, keepdims=True)
        oh = iota == ii
        ncx = jnp.sum(jnp.where(oh, x, 0.0), axis=1, keepdims=True)
        ncy = jnp.sum(jnp.where(oh, y, 0.0), axis=1, keepdims=True)
        ncz = jnp.sum(jnp.where(oh, z, 0.0), axis=1, keepdims=True)
        return ncx, ncy, ncz

    c0 = (x[:, 0:1], y[:, 0:1], z[:, 0:1])
    jax.lax.fori_loop(0, npoint, step, c0)


def _fps(xyz_planes, npoint):
    b, _, n = xyz_planes.shape
    return pl.pallas_call(
        functools.partial(_fps_body, npoint=npoint, n=n),
        out_shape=jax.ShapeDtypeStruct((b, npoint, 3), F32),
        scratch_shapes=[pltpu.VMEM((b, n), F32)],
    )(xyz_planes)


def _feat1_body(xyzc_ref, wsum_ref, out_ref):
    out_ref[0] = jnp.dot(xyzc_ref[0], wsum_ref[:, :],
                         preferred_element_type=F32)


def _feat2_body(xyzc_ref, pts_ref, wa_ref, wp_ref, out_ref):
    out_ref[0] = (
        jnp.dot(xyzc_ref[0], wa_ref[:, :], preferred_element_type=F32)
        + jnp.dot(pts_ref[0], wp_ref[:, :], preferred_element_type=F32))


def _feat1(xyzc, wsum):
    b, n, _ = xyzc.shape
    cdim = wsum.shape[1]
    return pl.pallas_call(
        _feat1_body,
        grid=(b,),
        in_specs=[
            pl.BlockSpec((1, n, 3), lambda i: (i, 0, 0)),
            pl.BlockSpec((3, cdim), lambda i: (0, 0)),
        ],
        out_specs=pl.BlockSpec((1, n, cdim), lambda i: (i, 0, 0)),
        out_shape=jax.ShapeDtypeStruct((b, n, cdim), F32),
    )(xyzc, wsum)


def _feat2(xyzc, pts, wa, wp):
    b, n, _ = xyzc.shape
    pdim = wp.shape[0]
    cdim = wa.shape[1]
    return pl.pallas_call(
        _feat2_body,
        grid=(b,),
        in_specs=[
            pl.BlockSpec((1, n, 3), lambda i: (i, 0, 0)),
            pl.BlockSpec((1, n, pdim), lambda i: (i, 0, 0)),
            pl.BlockSpec((3, cdim), lambda i: (0, 0)),
            pl.BlockSpec((pdim, cdim), lambda i: (0, 0)),
        ],
        out_specs=pl.BlockSpec((1, n, cdim), lambda i: (i, 0, 0)),
        out_shape=jax.ShapeDtypeStruct((b, n, cdim), F32),
    )(xyzc, pts, wa, wp)


def _ballidx_body(xyz4_ref, cenc_ref, out_ref, idxc_ref, off_ref,
                  *, nchunk, nsample, r2):
    k = pl.program_id(2)
    cen = cenc_ref[0]
    s = cen.shape[0]

    @pl.when(k == 0)
    def _():
        idxc_ref[:, :] = jnp.zeros((s, nsample), F32)
        off_ref[:, :] = jnp.zeros((s, 1), F32)

    cx = cen[:, 0:1]
    cy = cen[:, 1:2]
    cz = cen[:, 2:3]
    rit = jax.lax.broadcasted_iota(jnp.int32, (128, 128), 0)
    cit = jax.lax.broadcasted_iota(jnp.int32, (128, 128), 1)
    ut = (rit <= cit).astype(F32)

    x = xyz4_ref[0, 0, 0, 0:1, :]
    y = xyz4_ref[0, 1, 0, 0:1, :]
    z = xyz4_ref[0, 2, 0, 0:1, :]
    d = (cx - x) ** 2 + (cy - y) ** 2 + (cz - z) ** 2
    inb = d <= r2
    pc = (jnp.dot(inb.astype(F32), ut, preferred_element_type=F32)
          + off_ref[:, :])
    sel = jnp.logical_and(inb, pc <= float(nsample))
    jv = (jax.lax.broadcasted_iota(jnp.int32, (1, 128), 1)
          + k * 128 + 1).astype(F32)
    v = jnp.where(sel, jv, 0.0)
    cols = [jnp.sum(jnp.where(pc == float(q + 1), v, 0.0),
                    axis=1, keepdims=True)
            for q in range(nsample)]
    idxc_ref[:, :] = idxc_ref[:, :] + jnp.concatenate(cols, axis=1)
    off_ref[:, :] = pc[:, 127:128]

    @pl.when(k == nchunk - 1)
    def _():
        out_ref[0] = idxc_ref[:, :] - 1.0

    return


def _ballidx(xyz4, cenc, *, nsample, r2, s_tile):
    b, n3, nchunk = xyz4.shape[:3]
    s_total = cenc.shape[1]
    return pl.pallas_call(
        functools.partial(_ballidx_body, nchunk=nchunk, nsample=nsample,
                          r2=r2),
        grid=(b, s_total // s_tile, nchunk),
        in_specs=[
            pl.BlockSpec((1, 3, 1, 1, 128), lambda i, t, k: (i, 0, k, 0, 0)),
            pl.BlockSpec((1, s_tile, 3), lambda i, t, k: (i, t, 0)),
        ],
        out_specs=pl.BlockSpec((1, s_tile, nsample),
                               lambda i, t, k: (i, t, 0)),
        out_shape=jax.ShapeDtypeStruct((b, s_total, nsample), F32),
        scratch_shapes=[pltpu.VMEM((s_tile, nsample), F32),
                        pltpu.VMEM((s_tile, 1), F32)],
    )(xyz4, cenc)


def _gmax_body(idx_ref, f_ref, cenc_ref, wa_ref, b_ref, out_ref,
               *, s_tile, nsample):
    rows = []
    for s in range(s_tile):
        i0 = idx_ref[0, s, 0]
        row = None
        for q in range(nsample):
            iv = idx_ref[0, s, q]
            ivs = jnp.where(iv < 0, i0, iv)
            fr = f_ref[0, pl.ds(ivs, 1), :]
            row = fr if row is None else jnp.maximum(row, fr)
        rows.append(row)
    acc = jnp.concatenate(rows, axis=0)
    cen = cenc_ref[0]
    a = jnp.dot(cen, wa_ref[:, :], preferred_element_type=F32)
    out_ref[0] = jnp.maximum(acc - a + b_ref[0:1, :], 0.0)


def _gmax(idx, feat, cenc, wa, bvec, *, s_tile):
    b, s_total, nsample = idx.shape
    n = feat.shape[1]
    cdim = feat.shape[2]
    return pl.pallas_call(
        functools.partial(_gmax_body, s_tile=s_tile, nsample=nsample),
        grid=(b, s_total // s_tile),
        in_specs=[
            pl.BlockSpec((1, s_tile, nsample), lambda i, t: (i, t, 0),
                         memory_space=pltpu.SMEM),
            pl.BlockSpec((1, n, cdim), lambda i, t: (i, 0, 0)),
            pl.BlockSpec((1, s_tile, 3), lambda i, t: (i, t, 0)),
            pl.BlockSpec((3, cdim), lambda i, t: (0, 0)),
            pl.BlockSpec((1, cdim), lambda i, t: (0, 0)),
        ],
        out_specs=pl.BlockSpec((1, s_tile, cdim), lambda i, t: (i, t, 0)),
        out_shape=jax.ShapeDtypeStruct((b, s_total, cdim), F32),
    )(idx, feat, cenc, wa, bvec)


def _sa1(xyz4, xyzc, cenc, wsum, wa, bvec, *, nsample, r2, s_tile):
    feat = _feat1(xyzc, wsum)
    idx = _ballidx(xyz4, cenc, nsample=nsample, r2=r2, s_tile=256)
    return _gmax(idx.astype(jnp.int32), feat, cenc, wa, bvec, s_tile=32)


def _sa2(xyz4, xyzc, cenc, pts, wa, wp, bvec, *, nsample, r2, s_tile):
    feat = _feat2(xyzc, pts, wa, wp)
    idx = _ballidx(xyz4, cenc, nsample=nsample, r2=r2, s_tile=s_tile)
    return _gmax(idx.astype(jnp.int32), feat, cenc, wa, bvec, s_tile=32)


def _sa3fp3_body(cenc_ref, pts_ref, w3a_ref, w3p_ref, b3_ref,
                 wf3p_ref, wf3i_ref, bf3_ref, l3_ref, out_ref):
    pts = pts_ref[0]
    feat = jnp.maximum(
        jnp.dot(cenc_ref[0], w3a_ref[:, :], preferred_element_type=F32)
        + jnp.dot(pts, w3p_ref[:, :], preferred_element_type=F32)
        + b3_ref[0:1, :], 0.0)
    l3 = jnp.max(feat, axis=0, keepdims=True)
    l3_ref[0] = l3
    interp = jnp.dot(l3, wf3i_ref[:, :], preferred_element_type=F32)
    out_ref[0] = jnp.maximum(
        jnp.dot(pts, wf3p_ref[:, :], preferred_element_type=F32)
        + interp + bf3_ref[0:1, :], 0.0)


def _sa3fp3(cenc2, l2p, w3a, w3p, b3, wf3p, wf3i, bf3):
    b, s, _ = cenc2.shape
    pdim = l2p.shape[2]
    c3 = w3a.shape[1]
    cf = wf3p.shape[1]
    return pl.pallas_call(
        _sa3fp3_body,
        grid=(b,),
        in_specs=[
            pl.BlockSpec((1, s, 3), lambda i: (i, 0, 0)),
            pl.BlockSpec((1, s, pdim), lambda i: (i, 0, 0)),
            pl.BlockSpec((3, c3), lambda i: (0, 0)),
            pl.BlockSpec((pdim, c3), lambda i: (0, 0)),
            pl.BlockSpec((1, c3), lambda i: (0, 0)),
            pl.BlockSpec((pdim, cf), lambda i: (0, 0)),
            pl.BlockSpec((c3, cf), lambda i: (0, 0)),
            pl.BlockSpec((1, cf), lambda i: (0, 0)),
        ],
        out_specs=[
            pl.BlockSpec((1, 1, c3), lambda i: (i, 0, 0)),
            pl.BlockSpec((1, s, cf), lambda i: (i, 0, 0)),
        ],
        out_shape=[
            jax.ShapeDtypeStruct((b, 1, c3), F32),
            jax.ShapeDtypeStruct((b, s, cf), F32),
        ],
    )(cenc2, l2p, w3a, w3p, b3, wf3p, wf3i, bf3)


def _knn3_interp(d, pts):
    r, s2 = d.shape
    iota = jax.lax.broadcasted_iota(jnp.int32, (r, s2), 1)
    dd = d
    ohs = []
    ms = []
    for _ in range(3):
        m = jnp.min(dd, axis=1, keepdims=True)
        ii = jnp.min(jnp.where(dd == m, iota, s2), axis=1, keepdims=True)
        oh = iota == ii
        ms.append(m)
        ohs.append(oh)
        dd = jnp.where(oh, 3e38, dd)
    r1 = 1.0 / (ms[0] + 1e-8)
    r2 = 1.0 / (ms[1] + 1e-8)
    r3 = 1.0 / (ms[2] + 1e-8)
    norm = r1 + r2 + r3
    wh = (jnp.where(ohs[0], r1 / norm, 0.0)
          + jnp.where(ohs[1], r2 / norm, 0.0)
          + jnp.where(ohs[2], r3 / norm, 0.0))
    return jnp.dot(wh, pts, preferred_element_type=F32)


def _fp2_body(cen2_ref, cenc1_ref, l1p_ref, l2pn_ref, wp_ref, wi_ref,
              b_ref, out_ref):
    c1 = cenc1_ref[0]
    x2 = cen2_ref[0, 0:1, :]
    y2 = cen2_ref[0, 1:2, :]
    z2 = cen2_ref[0, 2:3, :]
    d = ((c1[:, 0:1] - x2) ** 2 + (c1[:, 1:2] - y2) ** 2
         + (c1[:, 2:3] - z2) ** 2)
    interp = _knn3_interp(d, l2pn_ref[0])
    out_ref[0] = jnp.maximum(
        jnp.dot(l1p_ref[0], wp_ref[:, :], preferred_element_type=F32)
        + jnp.dot(interp, wi_ref[:, :], preferred_element_type=F32)
        + b_ref[0:1, :], 0.0)


def _fp2(cen2, cenc1, l1p, l2pn, wp, wi, bvec):
    b, _, s2 = cen2.shape
    n1 = cenc1.shape[1]
    pdim = wp.shape[0]
    idim = wi.shape[0]
    cdim = wp.shape[1]
    return pl.pallas_call(
        _fp2_body,
        grid=(b,),
        in_specs=[
            pl.BlockSpec((1, 3, s2), lambda i: (i, 0, 0)),
            pl.BlockSpec((1, n1, 3), lambda i: (i, 0, 0)),
            pl.BlockSpec((1, n1, pdim), lambda i: (i, 0, 0)),
            pl.BlockSpec((1, s2, idim), lambda i: (i, 0, 0)),
            pl.BlockSpec((pdim, cdim), lambda i: (0, 0)),
            pl.BlockSpec((idim, cdim), lambda i: (0, 0)),
            pl.BlockSpec((1, cdim), lambda i: (0, 0)),
        ],
        out_specs=pl.BlockSpec((1, n1, cdim), lambda i: (i, 0, 0)),
        out_shape=jax.ShapeDtypeStruct((b, n1, cdim), F32),
    )(cen2, cenc1, l1p, l2pn, wp, wi, bvec)


def _fp1_body(cen1_ref, xyzc_ref, cls_ref, l1pn_ref, wc16_ref, wx_ref,
              wi_ref, b_ref, wcls_ref, bcls_ref, out_ref):
    c0 = xyzc_ref[0]
    x1 = cen1_ref[0, 0:1, :]
    y1 = cen1_ref[0, 1:2, :]
    z1 = cen1_ref[0, 2:3, :]
    d = ((c0[:, 0:1] - x1) ** 2 + (c0[:, 1:2] - y1) ** 2
         + (c0[:, 2:3] - z1) ** 2)
    interp = _knn3_interp(d, l1pn_ref[0])
    cls_part = jnp.dot(cls_ref[0], wc16_ref[:, :],
                       preferred_element_type=F32)
    feat = jnp.maximum(
        jnp.dot(interp, wi_ref[:, :], preferred_element_type=F32)
        + jnp.dot(c0, wx_ref[:, :], preferred_element_type=F32)
        + cls_part + b_ref[0:1, :], 0.0)
    logits = (jnp.dot(feat, wcls_ref[:, :], preferred_element_type=F32)
              + bcls_ref[0:1, :])
    m = jnp.max(logits, axis=1, keepdims=True)
    sh = logits - m
    out_ref[0] = sh - jnp.log(jnp.sum(jnp.exp(sh), axis=1, keepdims=True))


def _fp1(cen1, xyzc, cls3, l1pn, wc16, wx, wi, bvec, wcls, bcls, *, r_tile):
    b, _, s1 = cen1.shape
    n = xyzc.shape[1]
    idim = wi.shape[0]
    cdim = wi.shape[1]
    ncls = wcls.shape[1]
    return pl.pallas_call(
        _fp1_body,
        grid=(b, n // r_tile),
        in_specs=[
            pl.BlockSpec((1, 3, s1), lambda i, t: (i, 0, 0)),
            pl.BlockSpec((1, r_tile, 3), lambda i, t: (i, t, 0)),
            pl.BlockSpec((1, 1, 16), lambda i, t: (i, 0, 0)),
            pl.BlockSpec((1, s1, idim), lambda i, t: (i, 0, 0)),
            pl.BlockSpec((16, cdim), lambda i, t: (0, 0)),
            pl.BlockSpec((3, cdim), lambda i, t: (0, 0)),
            pl.BlockSpec((idim, cdim), lambda i, t: (0, 0)),
            pl.BlockSpec((1, cdim), lambda i, t: (0, 0)),
            pl.BlockSpec((cdim, ncls), lambda i, t: (0, 0)),
            pl.BlockSpec((1, ncls), lambda i, t: (0, 0)),
        ],
        out_specs=pl.BlockSpec((1, r_tile, ncls), lambda i, t: (i, t, 0)),
        out_shape=jax.ShapeDtypeStruct((b, n, ncls), F32),
    )(cen1, xyzc, cls3, l1pn, wc16, wx, wi, bvec, wcls, bcls)


def kernel(xyz, cls_label, W1, b1, W2, b2, W3, b3, Wf3, bf3, Wf2, bf2,
           Wf1, bf1, Wc, bc):
    b, _, n = xyz.shape
    xyzc = jnp.transpose(xyz, (0, 2, 1))
    xyz4 = xyz.reshape(b, 3, n // 128, 1, 128)

    cenc1 = _fps(xyz, 512)
    cen1 = jnp.transpose(cenc1, (0, 2, 1))

    l1p = _sa1(xyz4, xyzc, cenc1,
               W1[0:3] + W1[3:6], W1[0:3], b1.reshape(1, -1),
               nsample=32, r2=float(0.1 ** 2), s_tile=64)

    cenc2 = _fps(cen1, 128)
    cen2 = jnp.transpose(cenc2, (0, 2, 1))

    l2p = _sa2(cen1.reshape(b, 3, 4, 1, 128), cenc1, cenc2, l1p,
               W2[0:3], W2[3:], b2.reshape(1, -1),
               nsample=64, r2=float(0.3 ** 2), s_tile=64)

    l3, l2pn = _sa3fp3(cenc2, l2p, W3[0:3], W3[3:], b3.reshape(1, -1),
                       Wf3[0:256], Wf3[256:], bf3.reshape(1, -1))

    l1pn = _fp2(cen2, cenc1, l1p, l2pn,
                Wf2[0:128], Wf2[128:], bf2.reshape(1, -1))

    out = _fp1(cen1, xyzc, cls_label.reshape(b, 1, 16), l1pn,
               Wf1[0:16], Wf1[16:19] + Wf1[19:22], Wf1[22:],
               bf1.reshape(1, -1), Wc, bc.reshape(1, -1), r_tile=256)

    return out, jnp.transpose(l3, (0, 2, 1))


# speedup vs baseline: 9.5400x; 1.0496x over previous
import functools

import jax
import jax.numpy as jnp
from jax.experimental import pallas as pl
from jax.experimental.pallas import tpu as pltpu

F32 = jnp.float32
NEG = -1e30


def _fps_body(xyz_ref, out_ref, dist_ref, *, npoint, n):
    x = xyz_ref[:, 0, :]
    y = xyz_ref[:, 1, :]
    z = xyz_ref[:, 2, :]
    b = x.shape[0]
    dist_ref[:, :] = jnp.full((b, n), 1e10, F32)
    iota = jax.lax.broadcasted_iota(jnp.int32, (b, n), 1)

    def step(i, carry):
        cx, cy, cz = carry
        out_ref[:, pl.ds(i, 1), 0:1] = cx[:, :, None]
        out_ref[:, pl.ds(i, 1), 1:2] = cy[:, :, None]
        out_ref[:, pl.ds(i, 1), 2:3] = cz[:, :, None]
        d = (x - cx) ** 2 + (y - cy) ** 2 + (z - cz) ** 2
        dist = jnp.minimum(dist_ref[:, :], d)
        dist_ref[:, :] = dist
        m = jnp.max(dist, axis=1, keepdims=True)
        ii = jnp.min(jnp.where(dist == m, iota, n), axis=1, keepdims=True)
        oh = iota == ii
        ncx = jnp.sum(jnp.where(oh, x, 0.0), axis=1, keepdims=True)
        ncy = jnp.sum(jnp.where(oh, y, 0.0), axis=1, keepdims=True)
        ncz = jnp.sum(jnp.where(oh, z, 0.0), axis=1, keepdims=True)
        return ncx, ncy, ncz

    c0 = (x[:, 0:1], y[:, 0:1], z[:, 0:1])
    jax.lax.fori_loop(0, npoint, step, c0)


def _fps(xyz_planes, npoint):
    b, _, n = xyz_planes.shape
    return pl.pallas_call(
        functools.partial(_fps_body, npoint=npoint, n=n),
        out_shape=jax.ShapeDtypeStruct((b, npoint, 3), F32),
        scratch_shapes=[pltpu.VMEM((b, n), F32)],
    )(xyz_planes)


def _feat1_body(xyzc_ref, wsum_ref, out_ref):
    out_ref[0] = jnp.dot(xyzc_ref[0], wsum_ref[:, :],
                         preferred_element_type=F32)


def _feat2_body(xyzc_ref, pts_ref, wa_ref, wp_ref, out_ref):
    out_ref[0] = (
        jnp.dot(xyzc_ref[0], wa_ref[:, :], preferred_element_type=F32)
        + jnp.dot(pts_ref[0], wp_ref[:, :], preferred_element_type=F32))


def _feat1(xyzc, wsum):
    b, n, _ = xyzc.shape
    cdim = wsum.shape[1]
    return pl.pallas_call(
        _feat1_body,
        grid=(b,),
        in_specs=[
            pl.BlockSpec((1, n, 3), lambda i: (i, 0, 0)),
            pl.BlockSpec((3, cdim), lambda i: (0, 0)),
        ],
        out_specs=pl.BlockSpec((1, n, cdim), lambda i: (i, 0, 0)),
        out_shape=jax.ShapeDtypeStruct((b, n, cdim), F32),
    )(xyzc, wsum)


def _feat2(xyzc, pts, wa, wp):
    b, n, _ = xyzc.shape
    pdim = wp.shape[0]
    cdim = wa.shape[1]
    return pl.pallas_call(
        _feat2_body,
        grid=(b,),
        in_specs=[
            pl.BlockSpec((1, n, 3), lambda i: (i, 0, 0)),
            pl.BlockSpec((1, n, pdim), lambda i: (i, 0, 0)),
            pl.BlockSpec((3, cdim), lambda i: (0, 0)),
            pl.BlockSpec((pdim, cdim), lambda i: (0, 0)),
        ],
        out_specs=pl.BlockSpec((1, n, cdim), lambda i: (i, 0, 0)),
        out_shape=jax.ShapeDtypeStruct((b, n, cdim), F32),
    )(xyzc, pts, wa, wp)


def _ballidx_body(xyz4_ref, cenc_ref, out_ref, idxc_ref, off_ref,
                  *, nchunk, nsample, r2):
    k = pl.program_id(2)
    cen = cenc_ref[0]
    s = cen.shape[0]

    @pl.when(k == 0)
    def _():
        idxc_ref[:, :] = jnp.zeros((s, nsample), F32)
        off_ref[:, :] = jnp.zeros((s, 1), F32)

    cx = cen[:, 0:1]
    cy = cen[:, 1:2]
    cz = cen[:, 2:3]
    rit = jax.lax.broadcasted_iota(jnp.int32, (128, 128), 0)
    cit = jax.lax.broadcasted_iota(jnp.int32, (128, 128), 1)
    ut = (rit <= cit).astype(F32)

    x = xyz4_ref[0, 0, 0, 0:1, :]
    y = xyz4_ref[0, 1, 0, 0:1, :]
    z = xyz4_ref[0, 2, 0, 0:1, :]
    d = (cx - x) ** 2 + (cy - y) ** 2 + (cz - z) ** 2
    inb = d <= r2
    pc = (jnp.dot(inb.astype(F32), ut, preferred_element_type=F32)
          + off_ref[:, :])
    sel = jnp.logical_and(inb, pc <= float(nsample))
    jv = (jax.lax.broadcasted_iota(jnp.int32, (1, 128), 1)
          + k * 128 + 1).astype(F32)
    v = jnp.where(sel, jv, 0.0)
    cols = [jnp.sum(jnp.where(pc == float(q + 1), v, 0.0),
                    axis=1, keepdims=True)
            for q in range(nsample)]
    idxc_ref[:, :] = idxc_ref[:, :] + jnp.concatenate(cols, axis=1)
    off_ref[:, :] = pc[:, 127:128]

    @pl.when(k == nchunk - 1)
    def _():
        out_ref[0] = idxc_ref[:, :] - 1.0

    return


def _ballidx(xyz4, cenc, *, nsample, r2, s_tile):
    b, n3, nchunk = xyz4.shape[:3]
    s_total = cenc.shape[1]
    return pl.pallas_call(
        functools.partial(_ballidx_body, nchunk=nchunk, nsample=nsample,
                          r2=r2),
        grid=(b, s_total // s_tile, nchunk),
        in_specs=[
            pl.BlockSpec((1, 3, 1, 1, 128), lambda i, t, k: (i, 0, k, 0, 0)),
            pl.BlockSpec((1, s_tile, 3), lambda i, t, k: (i, t, 0)),
        ],
        out_specs=pl.BlockSpec((1, s_tile, nsample),
                               lambda i, t, k: (i, t, 0)),
        out_shape=jax.ShapeDtypeStruct((b, s_total, nsample), F32),
        scratch_shapes=[pltpu.VMEM((s_tile, nsample), F32),
                        pltpu.VMEM((s_tile, 1), F32)],
    )(xyz4, cenc)


def _gmax_body(idx_ref, f_ref, cenc_ref, wa_ref, b_ref, out_ref,
               *, s_tile, nsample):
    rows = []
    for s in range(s_tile):
        i0 = idx_ref[0, s, 0]
        row = None
        for q in range(nsample):
            iv = idx_ref[0, s, q]
            ivs = jnp.where(iv < 0, i0, iv)
            fr = f_ref[0, pl.ds(ivs, 1), :]
            row = fr if row is None else jnp.maximum(row, fr)
        rows.append(row)
    acc = jnp.concatenate(rows, axis=0)
    cen = cenc_ref[0]
    a = jnp.dot(cen, wa_ref[:, :], preferred_element_type=F32)
    out_ref[0] = jnp.maximum(acc - a + b_ref[0:1, :], 0.0)


def _gmax(idx, feat, cenc, wa, bvec, *, s_tile):
    b, s_total, nsample = idx.shape
    n = feat.shape[1]
    cdim = feat.shape[2]
    return pl.pallas_call(
        functools.partial(_gmax_body, s_tile=s_tile, nsample=nsample),
        grid=(b, s_total // s_tile),
        in_specs=[
            pl.BlockSpec((1, s_tile, nsample), lambda i, t: (i, t, 0),
                         memory_space=pltpu.SMEM),
            pl.BlockSpec((1, n, cdim), lambda i, t: (i, 0, 0)),
            pl.BlockSpec((1, s_tile, 3), lambda i, t: (i, t, 0)),
            pl.BlockSpec((3, cdim), lambda i, t: (0, 0)),
            pl.BlockSpec((1, cdim), lambda i, t: (0, 0)),
        ],
        out_specs=pl.BlockSpec((1, s_tile, cdim), lambda i, t: (i, t, 0)),
        out_shape=jax.ShapeDtypeStruct((b, s_total, cdim), F32),
    )(idx, feat, cenc, wa, bvec)


def _sa1(xyz4, xyzc, cenc, wsum, wa, bvec, *, nsample, r2, s_tile):
    feat = _feat1(xyzc, wsum)
    idx = _ballidx(xyz4, cenc, nsample=nsample, r2=r2, s_tile=512)
    return _gmax(idx.astype(jnp.int32), feat, cenc, wa, bvec, s_tile=64)


def _sa2(xyz4, xyzc, cenc, pts, wa, wp, bvec, *, nsample, r2, s_tile):
    feat = _feat2(xyzc, pts, wa, wp)
    idx = _ballidx(xyz4, cenc, nsample=nsample, r2=r2, s_tile=s_tile)
    return _gmax(idx.astype(jnp.int32), feat, cenc, wa, bvec, s_tile=32)


def _sa3fp3_body(cenc_ref, pts_ref, w3a_ref, w3p_ref, b3_ref,
                 wf3p_ref, wf3i_ref, bf3_ref, l3_ref, out_ref):
    pts = pts_ref[0]
    feat = jnp.maximum(
        jnp.dot(cenc_ref[0], w3a_ref[:, :], preferred_element_type=F32)
        + jnp.dot(pts, w3p_ref[:, :], preferred_element_type=F32)
        + b3_ref[0:1, :], 0.0)
    l3 = jnp.max(feat, axis=0, keepdims=True)
    l3_ref[0] = l3
    interp = jnp.dot(l3, wf3i_ref[:, :], preferred_element_type=F32)
    out_ref[0] = jnp.maximum(
        jnp.dot(pts, wf3p_ref[:, :], preferred_element_type=F32)
        + interp + bf3_ref[0:1, :], 0.0)


def _sa3fp3(cenc2, l2p, w3a, w3p, b3, wf3p, wf3i, bf3):
    b, s, _ = cenc2.shape
    pdim = l2p.shape[2]
    c3 = w3a.shape[1]
    cf = wf3p.shape[1]
    return pl.pallas_call(
        _sa3fp3_body,
        grid=(b,),
        in_specs=[
            pl.BlockSpec((1, s, 3), lambda i: (i, 0, 0)),
            pl.BlockSpec((1, s, pdim), lambda i: (i, 0, 0)),
            pl.BlockSpec((3, c3), lambda i: (0, 0)),
            pl.BlockSpec((pdim, c3), lambda i: (0, 0)),
            pl.BlockSpec((1, c3), lambda i: (0, 0)),
            pl.BlockSpec((pdim, cf), lambda i: (0, 0)),
            pl.BlockSpec((c3, cf), lambda i: (0, 0)),
            pl.BlockSpec((1, cf), lambda i: (0, 0)),
        ],
        out_specs=[
            pl.BlockSpec((1, 1, c3), lambda i: (i, 0, 0)),
            pl.BlockSpec((1, s, cf), lambda i: (i, 0, 0)),
        ],
        out_shape=[
            jax.ShapeDtypeStruct((b, 1, c3), F32),
            jax.ShapeDtypeStruct((b, s, cf), F32),
        ],
    )(cenc2, l2p, w3a, w3p, b3, wf3p, wf3i, bf3)


def _knn3_interp(d, pts):
    r, s2 = d.shape
    iota = jax.lax.broadcasted_iota(jnp.int32, (r, s2), 1)
    dd = d
    ohs = []
    ms = []
    for _ in range(3):
        m = jnp.min(dd, axis=1, keepdims=True)
        ii = jnp.min(jnp.where(dd == m, iota, s2), axis=1, keepdims=True)
        oh = iota == ii
        ms.append(m)
        ohs.append(oh)
        dd = jnp.where(oh, 3e38, dd)
    r1 = 1.0 / (ms[0] + 1e-8)
    r2 = 1.0 / (ms[1] + 1e-8)
    r3 = 1.0 / (ms[2] + 1e-8)
    norm = r1 + r2 + r3
    wh = (jnp.where(ohs[0], r1 / norm, 0.0)
          + jnp.where(ohs[1], r2 / norm, 0.0)
          + jnp.where(ohs[2], r3 / norm, 0.0))
    return jnp.dot(wh, pts, preferred_element_type=F32)


def _fp2_body(cen2_ref, cenc1_ref, l1p_ref, l2pn_ref, wp_ref, wi_ref,
              b_ref, out_ref):
    c1 = cenc1_ref[0]
    x2 = cen2_ref[0, 0:1, :]
    y2 = cen2_ref[0, 1:2, :]
    z2 = cen2_ref[0, 2:3, :]
    d = ((c1[:, 0:1] - x2) ** 2 + (c1[:, 1:2] - y2) ** 2
         + (c1[:, 2:3] - z2) ** 2)
    interp = _knn3_interp(d, l2pn_ref[0])
    out_ref[0] = jnp.maximum(
        jnp.dot(l1p_ref[0], wp_ref[:, :], preferred_element_type=F32)
        + jnp.dot(interp, wi_ref[:, :], preferred_element_type=F32)
        + b_ref[0:1, :], 0.0)


def _fp2(cen2, cenc1, l1p, l2pn, wp, wi, bvec):
    b, _, s2 = cen2.shape
    n1 = cenc1.shape[1]
    pdim = wp.shape[0]
    idim = wi.shape[0]
    cdim = wp.shape[1]
    return pl.pallas_call(
        _fp2_body,
        grid=(b,),
        in_specs=[
            pl.BlockSpec((1, 3, s2), lambda i: (i, 0, 0)),
            pl.BlockSpec((1, n1, 3), lambda i: (i, 0, 0)),
            pl.BlockSpec((1, n1, pdim), lambda i: (i, 0, 0)),
            pl.BlockSpec((1, s2, idim), lambda i: (i, 0, 0)),
            pl.BlockSpec((pdim, cdim), lambda i: (0, 0)),
            pl.BlockSpec((idim, cdim), lambda i: (0, 0)),
            pl.BlockSpec((1, cdim), lambda i: (0, 0)),
        ],
        out_specs=pl.BlockSpec((1, n1, cdim), lambda i: (i, 0, 0)),
        out_shape=jax.ShapeDtypeStruct((b, n1, cdim), F32),
    )(cen2, cenc1, l1p, l2pn, wp, wi, bvec)


def _fp1_body(cen1_ref, xyzc_ref, cls_ref, l1pn_ref, wc16_ref, wx_ref,
              wi_ref, b_ref, wcls_ref, bcls_ref, out_ref):
    c0 = xyzc_ref[0]
    x1 = cen1_ref[0, 0:1, :]
    y1 = cen1_ref[0, 1:2, :]
    z1 = cen1_ref[0, 2:3, :]
    d = ((c0[:, 0:1] - x1) ** 2 + (c0[:, 1:2] - y1) ** 2
         + (c0[:, 2:3] - z1) ** 2)
    interp = _knn3_interp(d, l1pn_ref[0])
    cls_part = jnp.dot(cls_ref[0], wc16_ref[:, :],
                       preferred_element_type=F32)
    feat = jnp.maximum(
        jnp.dot(interp, wi_ref[:, :], preferred_element_type=F32)
        + jnp.dot(c0, wx_ref[:, :], preferred_element_type=F32)
        + cls_part + b_ref[0:1, :], 0.0)
    logits = (jnp.dot(feat, wcls_ref[:, :], preferred_element_type=F32)
              + bcls_ref[0:1, :])
    m = jnp.max(logits, axis=1, keepdims=True)
    sh = logits - m
    out_ref[0] = sh - jnp.log(jnp.sum(jnp.exp(sh), axis=1, keepdims=True))


def _fp1(cen1, xyzc, cls3, l1pn, wc16, wx, wi, bvec, wcls, bcls, *, r_tile):
    b, _, s1 = cen1.shape
    n = xyzc.shape[1]
    idim = wi.shape[0]
    cdim = wi.shape[1]
    ncls = wcls.shape[1]
    return pl.pallas_call(
        _fp1_body,
        grid=(b, n // r_tile),
        in_specs=[
            pl.BlockSpec((1, 3, s1), lambda i, t: (i, 0, 0)),
            pl.BlockSpec((1, r_tile, 3), lambda i, t: (i, t, 0)),
            pl.BlockSpec((1, 1, 16), lambda i, t: (i, 0, 0)),
            pl.BlockSpec((1, s1, idim), lambda i, t: (i, 0, 0)),
            pl.BlockSpec((16, cdim), lambda i, t: (0, 0)),
            pl.BlockSpec((3, cdim), lambda i, t: (0, 0)),
            pl.BlockSpec((idim, cdim), lambda i, t: (0, 0)),
            pl.BlockSpec((1, cdim), lambda i, t: (0, 0)),
            pl.BlockSpec((cdim, ncls), lambda i, t: (0, 0)),
            pl.BlockSpec((1, ncls), lambda i, t: (0, 0)),
        ],
        out_specs=pl.BlockSpec((1, r_tile, ncls), lambda i, t: (i, t, 0)),
        out_shape=jax.ShapeDtypeStruct((b, n, ncls), F32),
    )(cen1, xyzc, cls3, l1pn, wc16, wx, wi, bvec, wcls, bcls)


def kernel(xyz, cls_label, W1, b1, W2, b2, W3, b3, Wf3, bf3, Wf2, bf2,
           Wf1, bf1, Wc, bc):
    b, _, n = xyz.shape
    xyzc = jnp.transpose(xyz, (0, 2, 1))
    xyz4 = xyz.reshape(b, 3, n // 128, 1, 128)

    cenc1 = _fps(xyz, 512)
    cen1 = jnp.transpose(cenc1, (0, 2, 1))

    l1p = _sa1(xyz4, xyzc, cenc1,
               W1[0:3] + W1[3:6], W1[0:3], b1.reshape(1, -1),
               nsample=32, r2=float(0.1 ** 2), s_tile=64)

    cenc2 = _fps(cen1, 128)
    cen2 = jnp.transpose(cenc2, (0, 2, 1))

    l2p = _sa2(cen1.reshape(b, 3, 4, 1, 128), cenc1, cenc2, l1p,
               W2[0:3], W2[3:], b2.reshape(1, -1),
               nsample=64, r2=float(0.3 ** 2), s_tile=64)

    l3, l2pn = _sa3fp3(cenc2, l2p, W3[0:3], W3[3:], b3.reshape(1, -1),
                       Wf3[0:256], Wf3[256:], bf3.reshape(1, -1))

    l1pn = _fp2(cen2, cenc1, l1p, l2pn,
                Wf2[0:128], Wf2[128:], bf2.reshape(1, -1))

    out = _fp1(cen1, xyzc, cls_label.reshape(b, 1, 16), l1pn,
               Wf1[0:16], Wf1[16:19] + Wf1[19:22], Wf1[22:],
               bf1.reshape(1, -1), Wc, bc.reshape(1, -1), r_tile=256)

    return out, jnp.transpose(l3, (0, 2, 1))
